# baseline (speedup 1.0000x reference)
import functools

import jax
import jax.numpy as jnp
from jax import lax
from jax.experimental import pallas as pl
from jax.experimental.pallas import tpu as pltpu
from jax.experimental.pallas import tpu_sc as plsc

V = 30522
D = 256
H = 4
DH = D // H
F = 1024
N = 8
TS = 512
K = 16
TK = 128

NBLK = 40
TOT = NBLK * 512
NW = 32
B_PER_W = TOT // NW
CH = 128
NCH = B_PER_W // CH


GCH = 64
NB = 6


def _sc_embed_gather(table, idx, tot):
    mesh = plsc.VectorSubcoreMesh(core_axis_name="c", subcore_axis_name="s")
    dh = D // 2
    bpw = tot // NW
    nch = bpw // GCH
    nb = min(NB, nch)

    @functools.partial(
        pl.kernel,
        mesh=mesh,
        out_type=jax.ShapeDtypeStruct((tot, dh), jnp.int32),
        scratch_types=(
            [pltpu.VMEM((bpw,), jnp.int32)]
            + [pltpu.VMEM((GCH, dh), jnp.int32) for _ in range(nb)]
            + [pltpu.SemaphoreType.DMA for _ in range(2 * nb)]
        ),
    )
    def k(table_hbm, idx_hbm, out_hbm, idx_v, *scr):
        bufs = scr[:nb]
        gsem = scr[nb:2 * nb]
        osem = scr[2 * nb:]
        wid = lax.axis_index("s") * 2 + lax.axis_index("c")
        base = wid * bpw
        pltpu.sync_copy(idx_hbm.at[pl.ds(base, bpw)], idx_v)
        gd = [None] * nch
        od = [None] * nch
        for c in range(nb):
            gd[c] = pltpu.async_copy(
                table_hbm.at[idx_v.at[pl.ds(c * GCH, GCH)]],
                bufs[c], gsem[c])
        for c in range(nch):
            b = c % nb
            gd[c].wait()
            od[c] = pltpu.async_copy(
                bufs[b], out_hbm.at[pl.ds(base + c * GCH, GCH)], osem[b])
            if c + nb < nch:
                od[c].wait()
                gd[c + nb] = pltpu.async_copy(
                    table_hbm.at[idx_v.at[pl.ds((c + nb) * GCH, GCH)]],
                    bufs[b], gsem[b])
        for c in range(max(0, nch - nb), nch):
            od[c].wait()

    return k(table, idx)


def _layernorm(x, g, b):
    m = jnp.mean(x, axis=-1, keepdims=True)
    v = jnp.mean((x - m) * (x - m), axis=-1, keepdims=True)
    return (x - m) / jnp.sqrt(v + 1e-5) * g + b


def _make_enc_body(chunk):
    nch = 512 // chunk

    def body(xg_ref, tok_ref, pos_ref, wq_ref, wk_ref, wv_ref, wo_ref,
             g1_ref, be1_ref, w1_ref, b1_ref, w2_ref, b2_ref, g2_ref, be2_ref,
             enc_ref, pool_ref):
        bf = jnp.bfloat16
        f32 = jnp.float32
        dot = functools.partial(lax.dot_general,
                                preferred_element_type=jnp.float32)
        x0 = xg_ref[...].astype(f32) * 16.0 + pos_ref[...]
        x0b = x0.astype(bf)
        tokr = tok_ref[0]
        mf = (tokr != 0).astype(jnp.float32)
        ones_col = jnp.ones((512, 1), jnp.float32)
        mk = ones_col @ mf
        mcol = lax.dot_general(mf, jnp.ones((1, 1), jnp.float32),
                               (((0,), (0,)), ((), ())))

        o = jnp.zeros((512, D), jnp.float32)
        for h in range(H):
            qh = dot(x0b, wq_ref[h], (((1,), (0,)), ((), ()))).astype(bf)
            kh = dot(x0b, wk_ref[h], (((1,), (0,)), ((), ()))).astype(bf)
            vh = dot(x0b, wv_ref[h], (((1,), (0,)), ((), ()))).astype(bf)
            ocs = []
            for c in range(nch):
                sl = slice(c * chunk, (c + 1) * chunk)
                sc = dot(qh[sl], kh[sl], (((1,), (1,)), ((), ()))) * 0.125
                sc = sc + (mk[sl, sl] - 1.0) * 1e9
                mx = jnp.max(sc, axis=-1, keepdims=True)
                e = jnp.exp(sc - mx)
                a = (e / jnp.sum(e, axis=-1, keepdims=True)).astype(bf)
                ocs.append(dot(a, vh[sl], (((1,), (0,)), ((), ()))))
            oh = ocs[0] if nch == 1 else jnp.concatenate(ocs, axis=0)
            o = o + dot(oh.astype(bf), wo_ref[h], (((1,), (0,)), ((), ())))

        x1 = _layernorm(x0 + o, g1_ref[...], be1_ref[...])
        ff = dot(jnp.maximum(
            dot(x1.astype(bf), w1_ref[...], (((1,), (0,)), ((), ())))
            + b1_ref[...], 0.0).astype(bf),
            w2_ref[...], (((1,), (0,)), ((), ()))) + b2_ref[...]
        x2 = _layernorm(x1 + ff, g2_ref[...], be2_ref[...])
        y = x2 * mcol
        enc_ref[...] = y

        yc = jnp.sum(y.reshape(4, 128, D), axis=1)
        cnt = jnp.sum(mcol.reshape(4, 128, 1), axis=1)
        cntb = jnp.broadcast_to(cnt, (4, D))
        pool_ref[...] = jnp.concatenate([yc, cntb], axis=0)

    return body


def _encode(xraw, tok3, posb, wqh, wkh, wvh, woh, g1, be1, w1, b1, w2, b2,
            g2, be2, *, chunk, grid_n, blk_off):
    full = lambda shape: pl.BlockSpec(shape, lambda i: tuple(0 for _ in shape))
    in_specs = [
        pl.BlockSpec((512, D), lambda i: (i, 0)),
        pl.BlockSpec((1, 1, 512), lambda i: (i + blk_off, 0, 0)),
        full((512, D)),
        full((H, D, DH)), full((H, D, DH)), full((H, D, DH)),
        full((H, DH, D)),
        full((1, D)), full((1, D)),
        full((D, F)), full((1, F)), full((F, D)), full((1, D)),
        full((1, D)), full((1, D)),
    ]
    out_specs = [
        pl.BlockSpec((512, D), lambda i: (i, 0)),
        pl.BlockSpec((8, D), lambda i: (i, 0)),
    ]
    return pl.pallas_call(
        _make_enc_body(chunk),
        grid=(grid_n,),
        in_specs=in_specs,
        out_specs=out_specs,
        out_shape=[
            jax.ShapeDtypeStruct((grid_n * 512, D), jnp.float32),
            jax.ShapeDtypeStruct((grid_n * 8, D), jnp.float32),
        ],
    )(xraw, tok3, posb, wqh, wkh, wvh, woh, g1, be1, w1, b1, w2, b2, g2, be2)


def _finish_body(pc_ref, pk_ref, cs_ref, use_ref, ck_ref, sel_ref):
    for n in range(N):
        b = 8 * n
        ctx_sum = (pc_ref[b + 0:b + 1, :] + pc_ref[b + 1:b + 2, :]
                   + pc_ref[b + 2:b + 3, :] + pc_ref[b + 3:b + 4, :])
        ctx_cnt = (pc_ref[b + 4:b + 5, :] + pc_ref[b + 5:b + 6, :]
                   + pc_ref[b + 6:b + 7, :] + pc_ref[b + 7:b + 8, :])
        ctx_use = ctx_sum / ctx_cnt
        ksum = jnp.concatenate(
            [pk_ref[8 * (4 * n + j):8 * (4 * n + j) + 4, :] for j in range(4)],
            axis=0)
        kcnt = jnp.concatenate(
            [pk_ref[8 * (4 * n + j) + 4:8 * (4 * n + j) + 8, :]
             for j in range(4)], axis=0)
        kuse = ksum / kcnt
        ckn = lax.dot_general(kuse, ctx_use, (((1,), (1,)), ((), ())))
        ck_ref[16 * n:16 * (n + 1), :] = jnp.broadcast_to(ckn, (16, 128))
        mx = jnp.max(ckn)
        ids = lax.broadcasted_iota(jnp.int32, (16, 1), 0)
        amx = jnp.min(jnp.where(ckn == mx, ids, K))
        sel_ref[n] = jnp.where(use_ref[0] != 0, cs_ref[n], amx)


def _finish(pooled_ctx, pooled_know, cs_ids, use_cs):
    return pl.pallas_call(
        _finish_body,
        in_specs=[
            pl.BlockSpec(memory_space=pltpu.VMEM),
            pl.BlockSpec(memory_space=pltpu.VMEM),
            pl.BlockSpec(memory_space=pltpu.SMEM),
            pl.BlockSpec(memory_space=pltpu.SMEM),
        ],
        out_specs=[
            pl.BlockSpec(memory_space=pltpu.VMEM),
            pl.BlockSpec(memory_space=pltpu.SMEM),
        ],
        out_shape=[
            jax.ShapeDtypeStruct((N * K, 128), jnp.float32),
            jax.ShapeDtypeStruct((N,), jnp.int32),
        ],
    )(pooled_ctx, pooled_know, cs_ids, use_cs)


def _select_body(sel_ref, cs_ref, ctx_ref, kt_ref, st_ref, fe_ref, ft_ref):
    fe_ref[0:TK, :] = cs_ref[...]
    fe_ref[TK:TK + TS, :] = ctx_ref[...]
    ft_ref[0, 0, 0:TK] = kt_ref[0, 0]
    ft_ref[0, 0, TK:TK + TS] = st_ref[0, 0]


def _select(enc_know, enc_ctx, ktok3, stok3, sel):
    grid_spec = pltpu.PrefetchScalarGridSpec(
        num_scalar_prefetch=1,
        grid=(N,),
        in_specs=[
            pl.BlockSpec((TK, D), lambda n, sel: (n * K + sel[n], 0)),
            pl.BlockSpec((TS, D), lambda n, sel: (n, 0)),
            pl.BlockSpec((1, 1, TK), lambda n, sel: (n * K + sel[n], 0, 0)),
            pl.BlockSpec((1, 1, TS), lambda n, sel: (n, 0, 0)),
        ],
        out_specs=[
            pl.BlockSpec((TK + TS, D), lambda n, sel: (n, 0)),
            pl.BlockSpec((1, 1, TK + TS), lambda n, sel: (n, 0, 0)),
        ],
    )
    return pl.pallas_call(
        _select_body,
        grid_spec=grid_spec,
        out_shape=[
            jax.ShapeDtypeStruct((N * (TK + TS), D), jnp.float32),
            jax.ShapeDtypeStruct((N, 1, TK + TS), jnp.int32),
        ],
    )(sel, enc_know, enc_ctx, ktok3, stok3)


def kernel(src_tokens, know_tokens, ck_mask, cs_ids, use_cs_ids, emb, pos,
           Wq, Wk, Wv, Wo, ln1_g, ln1_b, ln2_g, ln2_b, W1, b1, W2, b2):
    del ck_mask
    src32 = src_tokens.astype(jnp.int32)
    know32 = know_tokens.astype(jnp.int32)
    tok_flat = jnp.concatenate([src32.reshape(-1), know32.reshape(-1)])

    emb_i32 = lax.bitcast_convert_type(
        emb.astype(jnp.bfloat16).reshape(V, D // 2, 2), jnp.int32)
    xk = lax.bitcast_convert_type(
        _sc_embed_gather(emb_i32, know32.reshape(-1), 32 * 512),
        jnp.bfloat16).reshape(32 * 512, D)
    xc = lax.bitcast_convert_type(
        _sc_embed_gather(emb_i32, src32.reshape(-1), N * 512),
        jnp.bfloat16).reshape(N * 512, D)

    tok3 = tok_flat.reshape(NBLK, 1, 512)
    pos_ctx = pos
    pos_know = jnp.tile(pos[:TK], (4, 1))
    bf = jnp.bfloat16
    wqh = Wq.reshape(D, H, DH).transpose(1, 0, 2).astype(bf)
    wkh = Wk.reshape(D, H, DH).transpose(1, 0, 2).astype(bf)
    wvh = Wv.reshape(D, H, DH).transpose(1, 0, 2).astype(bf)
    woh = Wo.reshape(H, DH, D).astype(bf)
    g1 = ln1_g.reshape(1, D)
    be1 = ln1_b.reshape(1, D)
    g2 = ln2_g.reshape(1, D)
    be2 = ln2_b.reshape(1, D)
    b1r = b1.reshape(1, F)
    b2r = b2.reshape(1, D)

    w1b = W1.astype(bf)
    w2b = W2.astype(bf)
    enc_know, pooled_know = _encode(
        xk, tok3, pos_know, wqh, wkh, wvh, woh, g1, be1, w1b, b1r, w2b, b2r,
        g2, be2, chunk=TK, grid_n=32, blk_off=N)
    enc_ctx, pooled_ctx = _encode(
        xc, tok3, pos_ctx, wqh, wkh, wvh, woh, g1, be1, w1b, b1r, w2b, b2r,
        g2, be2, chunk=512, grid_n=N, blk_off=0)

    use_cs = jnp.asarray(use_cs_ids, jnp.int32).reshape(1)
    cs32 = cs_ids.astype(jnp.int32)
    ck128, sel = _finish(pooled_ctx, pooled_know, cs32, use_cs)
    ck_attn = ck128[:, 0].reshape(N, K)

    ktok3 = know32.reshape(N * K, 1, TK)
    stok3 = src32.reshape(N, 1, TS)
    full_enc_flat, full_tok = _select(enc_know, enc_ctx, ktok3, stok3, sel)

    full_enc = full_enc_flat.reshape(N, TK + TS, D)
    full_mask = full_tok.reshape(N, TK + TS) != 0
    return full_enc, full_mask, ck_attn


# speedup vs baseline: 1.7622x; 1.7622x over previous
import functools

import jax
import jax.numpy as jnp
from jax import lax
from jax.experimental import pallas as pl
from jax.experimental.pallas import tpu as pltpu
from jax.experimental.pallas import tpu_sc as plsc

V = 30522
D = 256
H = 4
DH = D // H
F = 1024
N = 8
TS = 512
K = 16
TK = 128

NBLK = 40
TOT = NBLK * 512
NW = 32
B_PER_W = TOT // NW
CH = 128
NCH = B_PER_W // CH


GCH = 64
NB = 6


def _sc_embed_gather(table, idx, tot):
    mesh = plsc.VectorSubcoreMesh(core_axis_name="c", subcore_axis_name="s")
    bpw = tot // NW
    nch = bpw // GCH
    nb = min(NB, nch)

    @functools.partial(
        pl.kernel,
        mesh=mesh,
        out_type=jax.ShapeDtypeStruct((tot, D), jnp.float32),
        scratch_types=(
            [pltpu.VMEM((bpw,), jnp.int32)]
            + [pltpu.VMEM((GCH, D), jnp.float32) for _ in range(nb)]
            + [pltpu.SemaphoreType.DMA for _ in range(2 * nb)]
        ),
    )
    def k(table_hbm, idx_hbm, out_hbm, idx_v, *scr):
        bufs = scr[:nb]
        gsem = scr[nb:2 * nb]
        osem = scr[2 * nb:]
        wid = lax.axis_index("s") * 2 + lax.axis_index("c")
        base = wid * bpw
        pltpu.sync_copy(idx_hbm.at[pl.ds(base, bpw)], idx_v)
        gd = [None] * nch
        od = [None] * nch
        for c in range(nb):
            gd[c] = pltpu.async_copy(
                table_hbm.at[idx_v.at[pl.ds(c * GCH, GCH)]],
                bufs[c], gsem[c])
        for c in range(nch):
            b = c % nb
            gd[c].wait()
            od[c] = pltpu.async_copy(
                bufs[b], out_hbm.at[pl.ds(base + c * GCH, GCH)], osem[b])
            if c + nb < nch:
                od[c].wait()
                gd[c + nb] = pltpu.async_copy(
                    table_hbm.at[idx_v.at[pl.ds((c + nb) * GCH, GCH)]],
                    bufs[b], gsem[b])
        for c in range(max(0, nch - nb), nch):
            od[c].wait()

    return k(table, idx)


def _layernorm(x, g, b):
    m = jnp.mean(x, axis=-1, keepdims=True)
    v = jnp.mean((x - m) * (x - m), axis=-1, keepdims=True)
    return (x - m) / jnp.sqrt(v + 1e-5) * g + b


def _make_enc_body(chunk):
    nch = 512 // chunk

    def body(xg_ref, tok_ref, pos_ref, wq_ref, wk_ref, wv_ref, wo_ref,
             g1_ref, be1_ref, w1_ref, b1_ref, w2_ref, b2_ref, g2_ref, be2_ref,
             enc_ref, pool_ref):
        bf = jnp.bfloat16
        f32 = jnp.float32
        dot = functools.partial(lax.dot_general,
                                preferred_element_type=jnp.float32)
        x0 = xg_ref[...].astype(f32) * 16.0 + pos_ref[...]
        x0b = x0.astype(bf)
        tokr = tok_ref[0]
        mf = (tokr != 0).astype(jnp.float32)
        ones_col = jnp.ones((512, 1), jnp.float32)
        mk = ones_col @ mf
        mcol = lax.dot_general(mf, jnp.ones((1, 1), jnp.float32),
                               (((0,), (0,)), ((), ())))

        o = jnp.zeros((512, D), jnp.float32)
        for h in range(H):
            qh = dot(x0b, wq_ref[h], (((1,), (0,)), ((), ()))).astype(bf)
            kh = dot(x0b, wk_ref[h], (((1,), (0,)), ((), ()))).astype(bf)
            vh = dot(x0b, wv_ref[h], (((1,), (0,)), ((), ()))).astype(bf)
            ocs = []
            for c in range(nch):
                sl = slice(c * chunk, (c + 1) * chunk)
                sc = dot(qh[sl], kh[sl], (((1,), (1,)), ((), ()))) * 0.125
                sc = sc + (mk[sl, sl] - 1.0) * 1e9
                mx = jnp.max(sc, axis=-1, keepdims=True)
                e = jnp.exp(sc - mx)
                a = (e / jnp.sum(e, axis=-1, keepdims=True)).astype(bf)
                ocs.append(dot(a, vh[sl], (((1,), (0,)), ((), ()))))
            oh = ocs[0] if nch == 1 else jnp.concatenate(ocs, axis=0)
            o = o + dot(oh.astype(bf), wo_ref[h], (((1,), (0,)), ((), ())))

        x1 = _layernorm(x0 + o, g1_ref[...], be1_ref[...])
        ff = dot(jnp.maximum(
            dot(x1.astype(bf), w1_ref[...], (((1,), (0,)), ((), ())))
            + b1_ref[...], 0.0).astype(bf),
            w2_ref[...], (((1,), (0,)), ((), ()))) + b2_ref[...]
        x2 = _layernorm(x1 + ff, g2_ref[...], be2_ref[...])
        y = x2 * mcol
        enc_ref[...] = y

        yc = jnp.sum(y.reshape(4, 128, D), axis=1)
        cnt = jnp.sum(mcol.reshape(4, 128, 1), axis=1)
        cntb = jnp.broadcast_to(cnt, (4, D))
        pool_ref[...] = jnp.concatenate([yc, cntb], axis=0)

    return body


def _encode(xraw, tok3, posb, wqh, wkh, wvh, woh, g1, be1, w1, b1, w2, b2,
            g2, be2, *, chunk, grid_n, blk_off):
    full = lambda shape: pl.BlockSpec(shape, lambda i: tuple(0 for _ in shape))
    in_specs = [
        pl.BlockSpec((512, D), lambda i: (i, 0)),
        pl.BlockSpec((1, 1, 512), lambda i: (i + blk_off, 0, 0)),
        full((512, D)),
        full((H, D, DH)), full((H, D, DH)), full((H, D, DH)),
        full((H, DH, D)),
        full((1, D)), full((1, D)),
        full((D, F)), full((1, F)), full((F, D)), full((1, D)),
        full((1, D)), full((1, D)),
    ]
    out_specs = [
        pl.BlockSpec((512, D), lambda i: (i, 0)),
        pl.BlockSpec((8, D), lambda i: (i, 0)),
    ]
    return pl.pallas_call(
        _make_enc_body(chunk),
        grid=(grid_n,),
        in_specs=in_specs,
        out_specs=out_specs,
        out_shape=[
            jax.ShapeDtypeStruct((grid_n * 512, D), jnp.float32),
            jax.ShapeDtypeStruct((grid_n * 8, D), jnp.float32),
        ],
    )(xraw, tok3, posb, wqh, wkh, wvh, woh, g1, be1, w1, b1, w2, b2, g2, be2)


def _finish_body(pc_ref, pk_ref, cs_ref, use_ref, ck_ref, sel_ref):
    for n in range(N):
        b = 8 * n
        ctx_sum = (pc_ref[b + 0:b + 1, :] + pc_ref[b + 1:b + 2, :]
                   + pc_ref[b + 2:b + 3, :] + pc_ref[b + 3:b + 4, :])
        ctx_cnt = (pc_ref[b + 4:b + 5, :] + pc_ref[b + 5:b + 6, :]
                   + pc_ref[b + 6:b + 7, :] + pc_ref[b + 7:b + 8, :])
        ctx_use = ctx_sum / ctx_cnt
        ksum = jnp.concatenate(
            [pk_ref[8 * (4 * n + j):8 * (4 * n + j) + 4, :] for j in range(4)],
            axis=0)
        kcnt = jnp.concatenate(
            [pk_ref[8 * (4 * n + j) + 4:8 * (4 * n + j) + 8, :]
             for j in range(4)], axis=0)
        kuse = ksum / kcnt
        ckn = lax.dot_general(kuse, ctx_use, (((1,), (1,)), ((), ())))
        ck_ref[16 * n:16 * (n + 1), :] = jnp.broadcast_to(ckn, (16, 128))
        mx = jnp.max(ckn)
        ids = lax.broadcasted_iota(jnp.int32, (16, 1), 0)
        amx = jnp.min(jnp.where(ckn == mx, ids, K))
        sel_ref[n] = jnp.where(use_ref[0] != 0, cs_ref[n], amx)


def _finish(pooled_ctx, pooled_know, cs_ids, use_cs):
    return pl.pallas_call(
        _finish_body,
        in_specs=[
            pl.BlockSpec(memory_space=pltpu.VMEM),
            pl.BlockSpec(memory_space=pltpu.VMEM),
            pl.BlockSpec(memory_space=pltpu.SMEM),
            pl.BlockSpec(memory_space=pltpu.SMEM),
        ],
        out_specs=[
            pl.BlockSpec(memory_space=pltpu.VMEM),
            pl.BlockSpec(memory_space=pltpu.SMEM),
        ],
        out_shape=[
            jax.ShapeDtypeStruct((N * K, 128), jnp.float32),
            jax.ShapeDtypeStruct((N,), jnp.int32),
        ],
    )(pooled_ctx, pooled_know, cs_ids, use_cs)


def _select_body(sel_ref, cs_ref, ctx_ref, kt_ref, st_ref, fe_ref, ft_ref):
    fe_ref[0:TK, :] = cs_ref[...]
    fe_ref[TK:TK + TS, :] = ctx_ref[...]
    ft_ref[0, 0, 0:TK] = kt_ref[0, 0]
    ft_ref[0, 0, TK:TK + TS] = st_ref[0, 0]


def _select(enc_know, enc_ctx, ktok3, stok3, sel):
    grid_spec = pltpu.PrefetchScalarGridSpec(
        num_scalar_prefetch=1,
        grid=(N,),
        in_specs=[
            pl.BlockSpec((TK, D), lambda n, sel: (n * K + sel[n], 0)),
            pl.BlockSpec((TS, D), lambda n, sel: (n, 0)),
            pl.BlockSpec((1, 1, TK), lambda n, sel: (n * K + sel[n], 0, 0)),
            pl.BlockSpec((1, 1, TS), lambda n, sel: (n, 0, 0)),
        ],
        out_specs=[
            pl.BlockSpec((TK + TS, D), lambda n, sel: (n, 0)),
            pl.BlockSpec((1, 1, TK + TS), lambda n, sel: (n, 0, 0)),
        ],
    )
    return pl.pallas_call(
        _select_body,
        grid_spec=grid_spec,
        out_shape=[
            jax.ShapeDtypeStruct((N * (TK + TS), D), jnp.float32),
            jax.ShapeDtypeStruct((N, 1, TK + TS), jnp.int32),
        ],
    )(sel, enc_know, enc_ctx, ktok3, stok3)


def kernel(src_tokens, know_tokens, ck_mask, cs_ids, use_cs_ids, emb, pos,
           Wq, Wk, Wv, Wo, ln1_g, ln1_b, ln2_g, ln2_b, W1, b1, W2, b2):
    del ck_mask
    src32 = src_tokens.astype(jnp.int32)
    know32 = know_tokens.astype(jnp.int32)
    tok_flat = jnp.concatenate([src32.reshape(-1), know32.reshape(-1)])

    xk = _sc_embed_gather(emb, know32.reshape(-1), 32 * 512)
    xc = _sc_embed_gather(emb, src32.reshape(-1), N * 512)

    tok3 = tok_flat.reshape(NBLK, 1, 512)
    pos_ctx = pos
    pos_know = jnp.tile(pos[:TK], (4, 1))
    bf = jnp.bfloat16
    wqh = Wq.reshape(D, H, DH).transpose(1, 0, 2).astype(bf)
    wkh = Wk.reshape(D, H, DH).transpose(1, 0, 2).astype(bf)
    wvh = Wv.reshape(D, H, DH).transpose(1, 0, 2).astype(bf)
    woh = Wo.reshape(H, DH, D).astype(bf)
    g1 = ln1_g.reshape(1, D)
    be1 = ln1_b.reshape(1, D)
    g2 = ln2_g.reshape(1, D)
    be2 = ln2_b.reshape(1, D)
    b1r = b1.reshape(1, F)
    b2r = b2.reshape(1, D)

    w1b = W1.astype(bf)
    w2b = W2.astype(bf)
    enc_know, pooled_know = _encode(
        xk, tok3, pos_know, wqh, wkh, wvh, woh, g1, be1, w1b, b1r, w2b, b2r,
        g2, be2, chunk=TK, grid_n=32, blk_off=N)
    enc_ctx, pooled_ctx = _encode(
        xc, tok3, pos_ctx, wqh, wkh, wvh, woh, g1, be1, w1b, b1r, w2b, b2r,
        g2, be2, chunk=512, grid_n=N, blk_off=0)

    use_cs = jnp.asarray(use_cs_ids, jnp.int32).reshape(1)
    cs32 = cs_ids.astype(jnp.int32)
    ck128, sel = _finish(pooled_ctx, pooled_know, cs32, use_cs)
    ck_attn = ck128[:, 0].reshape(N, K)

    ktok3 = know32.reshape(N * K, 1, TK)
    stok3 = src32.reshape(N, 1, TS)
    full_enc_flat, full_tok = _select(enc_know, enc_ctx, ktok3, stok3, sel)

    full_enc = full_enc_flat.reshape(N, TK + TS, D)
    full_mask = full_tok.reshape(N, TK + TS) != 0
    return full_enc, full_mask, ck_attn


# speedup vs baseline: 1.8205x; 1.0331x over previous
import functools

import jax
import jax.numpy as jnp
from jax import lax
from jax.experimental import pallas as pl
from jax.experimental.pallas import tpu as pltpu
from jax.experimental.pallas import tpu_sc as plsc

V = 30522
D = 256
H = 4
DH = D // H
F = 1024
N = 8
TS = 512
K = 16
TK = 128

NBLK = 40
TOT = NBLK * 512
NW = 32
B_PER_W = TOT // NW
CH = 128
NCH = B_PER_W // CH


GCH = 64
NB = 6


def _sc_embed_gather(table, idx, tot):
    mesh = plsc.VectorSubcoreMesh(core_axis_name="c", subcore_axis_name="s")
    bpw = tot // NW
    nch = bpw // GCH
    nb = min(NB, nch)

    @functools.partial(
        pl.kernel,
        mesh=mesh,
        out_type=jax.ShapeDtypeStruct((tot, D), jnp.float32),
        scratch_types=(
            [pltpu.VMEM((bpw,), jnp.int32)]
            + [pltpu.VMEM((GCH, D), jnp.float32) for _ in range(nb)]
            + [pltpu.SemaphoreType.DMA for _ in range(2 * nb)]
        ),
    )
    def k(table_hbm, idx_hbm, out_hbm, idx_v, *scr):
        bufs = scr[:nb]
        gsem = scr[nb:2 * nb]
        osem = scr[2 * nb:]
        wid = lax.axis_index("s") * 2 + lax.axis_index("c")
        base = wid * bpw
        pltpu.sync_copy(idx_hbm.at[pl.ds(base, bpw)], idx_v)
        gd = [None] * nch
        od = [None] * nch
        for c in range(nb):
            gd[c] = pltpu.async_copy(
                table_hbm.at[idx_v.at[pl.ds(c * GCH, GCH)]],
                bufs[c], gsem[c])
        for c in range(nch):
            b = c % nb
            gd[c].wait()
            od[c] = pltpu.async_copy(
                bufs[b], out_hbm.at[pl.ds(base + c * GCH, GCH)], osem[b])
            if c + nb < nch:
                od[c].wait()
                gd[c + nb] = pltpu.async_copy(
                    table_hbm.at[idx_v.at[pl.ds((c + nb) * GCH, GCH)]],
                    bufs[b], gsem[b])
        for c in range(max(0, nch - nb), nch):
            od[c].wait()

    return k(table, idx)


def _layernorm(x, g, b):
    m = jnp.mean(x, axis=-1, keepdims=True)
    v = jnp.mean((x - m) * (x - m), axis=-1, keepdims=True)
    return (x - m) / jnp.sqrt(v + 1e-5) * g + b


def _make_enc_body(chunk):
    nch = 512 // chunk

    def body(xg_ref, tok_ref, pos_ref, wq_ref, wk_ref, wv_ref, wo_ref,
             g1_ref, be1_ref, w1_ref, b1_ref, w2_ref, b2_ref, g2_ref, be2_ref,
             enc_ref, pool_ref):
        bf = jnp.bfloat16
        f32 = jnp.float32
        dot = functools.partial(lax.dot_general,
                                preferred_element_type=jnp.float32)
        x0 = xg_ref[...].astype(f32) * 16.0 + pos_ref[...]
        x0b = x0.astype(bf)
        tokr = tok_ref[0]
        mf = (tokr != 0).astype(jnp.float32)
        ones_col = jnp.ones((512, 1), jnp.float32)
        mk = ones_col @ mf
        mcol = lax.dot_general(mf, jnp.ones((1, 1), jnp.float32),
                               (((0,), (0,)), ((), ())))

        cd = (((1,), (0,)), ((), ()))
        qf = dot(x0b, wq_ref[...], cd).astype(bf)
        kf = dot(x0b, wk_ref[...], cd).astype(bf)
        vf = dot(x0b, wv_ref[...], cd).astype(bf)
        ohs = []
        for h in range(H):
            hsl = slice(h * DH, (h + 1) * DH)
            qh, kh, vh = qf[:, hsl], kf[:, hsl], vf[:, hsl]
            ocs = []
            for c in range(nch):
                sl = slice(c * chunk, (c + 1) * chunk)
                sc = dot(qh[sl], kh[sl], (((1,), (1,)), ((), ()))) * 0.125
                e = jnp.exp(sc) * mk[sl, sl]
                a = (e / jnp.sum(e, axis=-1, keepdims=True)).astype(bf)
                ocs.append(dot(a, vh[sl], (((1,), (0,)), ((), ()))))
            ohs.append(ocs[0] if nch == 1 else jnp.concatenate(ocs, axis=0))
        oc = jnp.concatenate(ohs, axis=1).astype(bf)
        o = dot(oc, wo_ref[...], cd)

        x1 = _layernorm(x0 + o, g1_ref[...], be1_ref[...])
        ff = dot(jnp.maximum(
            dot(x1.astype(bf), w1_ref[...], (((1,), (0,)), ((), ())))
            + b1_ref[...], 0.0).astype(bf),
            w2_ref[...], (((1,), (0,)), ((), ()))) + b2_ref[...]
        x2 = _layernorm(x1 + ff, g2_ref[...], be2_ref[...])
        y = x2 * mcol
        enc_ref[...] = y

        yc = jnp.sum(y.reshape(4, 128, D), axis=1)
        cnt = jnp.sum(mcol.reshape(4, 128, 1), axis=1)
        cntb = jnp.broadcast_to(cnt, (4, D))
        pool_ref[...] = jnp.concatenate([yc, cntb], axis=0)

    return body


def _encode(xraw, tok3, posb, wqh, wkh, wvh, woh, g1, be1, w1, b1, w2, b2,
            g2, be2, *, chunk, grid_n, blk_off):
    full = lambda shape: pl.BlockSpec(shape, lambda i: tuple(0 for _ in shape))
    in_specs = [
        pl.BlockSpec((512, D), lambda i: (i, 0)),
        pl.BlockSpec((1, 1, 512), lambda i: (i + blk_off, 0, 0)),
        full((512, D)),
        full((D, D)), full((D, D)), full((D, D)), full((D, D)),
        full((1, D)), full((1, D)),
        full((D, F)), full((1, F)), full((F, D)), full((1, D)),
        full((1, D)), full((1, D)),
    ]
    out_specs = [
        pl.BlockSpec((512, D), lambda i: (i, 0)),
        pl.BlockSpec((8, D), lambda i: (i, 0)),
    ]
    return pl.pallas_call(
        _make_enc_body(chunk),
        grid=(grid_n,),
        in_specs=in_specs,
        out_specs=out_specs,
        out_shape=[
            jax.ShapeDtypeStruct((grid_n * 512, D), jnp.float32),
            jax.ShapeDtypeStruct((grid_n * 8, D), jnp.float32),
        ],
    )(xraw, tok3, posb, wqh, wkh, wvh, woh, g1, be1, w1, b1, w2, b2, g2, be2)


def _finish_body(pc_ref, pk_ref, cs_ref, use_ref, ck_ref, sel_ref):
    for n in range(N):
        b = 8 * n
        ctx_sum = (pc_ref[b + 0:b + 1, :] + pc_ref[b + 1:b + 2, :]
                   + pc_ref[b + 2:b + 3, :] + pc_ref[b + 3:b + 4, :])
        ctx_cnt = (pc_ref[b + 4:b + 5, :] + pc_ref[b + 5:b + 6, :]
                   + pc_ref[b + 6:b + 7, :] + pc_ref[b + 7:b + 8, :])
        ctx_use = ctx_sum / ctx_cnt
        ksum = jnp.concatenate(
            [pk_ref[8 * (4 * n + j):8 * (4 * n + j) + 4, :] for j in range(4)],
            axis=0)
        kcnt = jnp.concatenate(
            [pk_ref[8 * (4 * n + j) + 4:8 * (4 * n + j) + 8, :]
             for j in range(4)], axis=0)
        kuse = ksum / kcnt
        ckn = lax.dot_general(kuse, ctx_use, (((1,), (1,)), ((), ())))
        ck_ref[16 * n:16 * (n + 1), :] = jnp.broadcast_to(ckn, (16, 128))
        mx = jnp.max(ckn)
        ids = lax.broadcasted_iota(jnp.int32, (16, 1), 0)
        amx = jnp.min(jnp.where(ckn == mx, ids, K))
        sel_ref[n] = jnp.where(use_ref[0] != 0, cs_ref[n], amx)


def _finish(pooled_ctx, pooled_know, cs_ids, use_cs):
    return pl.pallas_call(
        _finish_body,
        in_specs=[
            pl.BlockSpec(memory_space=pltpu.VMEM),
            pl.BlockSpec(memory_space=pltpu.VMEM),
            pl.BlockSpec(memory_space=pltpu.SMEM),
            pl.BlockSpec(memory_space=pltpu.SMEM),
        ],
        out_specs=[
            pl.BlockSpec(memory_space=pltpu.VMEM),
            pl.BlockSpec(memory_space=pltpu.SMEM),
        ],
        out_shape=[
            jax.ShapeDtypeStruct((N * K, 128), jnp.float32),
            jax.ShapeDtypeStruct((N,), jnp.int32),
        ],
    )(pooled_ctx, pooled_know, cs_ids, use_cs)


def _select_body(sel_ref, cs_ref, ctx_ref, kt_ref, st_ref, fe_ref, ft_ref):
    fe_ref[0:TK, :] = cs_ref[...]
    fe_ref[TK:TK + TS, :] = ctx_ref[...]
    ft_ref[0, 0, 0:TK] = kt_ref[0, 0]
    ft_ref[0, 0, TK:TK + TS] = st_ref[0, 0]


def _select(enc_know, enc_ctx, ktok3, stok3, sel):
    grid_spec = pltpu.PrefetchScalarGridSpec(
        num_scalar_prefetch=1,
        grid=(N,),
        in_specs=[
            pl.BlockSpec((TK, D), lambda n, sel: (n * K + sel[n], 0)),
            pl.BlockSpec((TS, D), lambda n, sel: (n, 0)),
            pl.BlockSpec((1, 1, TK), lambda n, sel: (n * K + sel[n], 0, 0)),
            pl.BlockSpec((1, 1, TS), lambda n, sel: (n, 0, 0)),
        ],
        out_specs=[
            pl.BlockSpec((TK + TS, D), lambda n, sel: (n, 0)),
            pl.BlockSpec((1, 1, TK + TS), lambda n, sel: (n, 0, 0)),
        ],
    )
    return pl.pallas_call(
        _select_body,
        grid_spec=grid_spec,
        out_shape=[
            jax.ShapeDtypeStruct((N * (TK + TS), D), jnp.float32),
            jax.ShapeDtypeStruct((N, 1, TK + TS), jnp.int32),
        ],
    )(sel, enc_know, enc_ctx, ktok3, stok3)


def kernel(src_tokens, know_tokens, ck_mask, cs_ids, use_cs_ids, emb, pos,
           Wq, Wk, Wv, Wo, ln1_g, ln1_b, ln2_g, ln2_b, W1, b1, W2, b2):
    del ck_mask
    src32 = src_tokens.astype(jnp.int32)
    know32 = know_tokens.astype(jnp.int32)
    tok_flat = jnp.concatenate([src32.reshape(-1), know32.reshape(-1)])

    xk = _sc_embed_gather(emb, know32.reshape(-1), 32 * 512)
    xc = _sc_embed_gather(emb, src32.reshape(-1), N * 512)

    tok3 = tok_flat.reshape(NBLK, 1, 512)
    pos_ctx = pos
    pos_know = jnp.tile(pos[:TK], (4, 1))
    bf = jnp.bfloat16
    wqh = Wq.astype(bf)
    wkh = Wk.astype(bf)
    wvh = Wv.astype(bf)
    woh = Wo.astype(bf)
    g1 = ln1_g.reshape(1, D)
    be1 = ln1_b.reshape(1, D)
    g2 = ln2_g.reshape(1, D)
    be2 = ln2_b.reshape(1, D)
    b1r = b1.reshape(1, F)
    b2r = b2.reshape(1, D)

    w1b = W1.astype(bf)
    w2b = W2.astype(bf)
    enc_know, pooled_know = _encode(
        xk, tok3, pos_know, wqh, wkh, wvh, woh, g1, be1, w1b, b1r, w2b, b2r,
        g2, be2, chunk=TK, grid_n=32, blk_off=N)
    enc_ctx, pooled_ctx = _encode(
        xc, tok3, pos_ctx, wqh, wkh, wvh, woh, g1, be1, w1b, b1r, w2b, b2r,
        g2, be2, chunk=512, grid_n=N, blk_off=0)

    use_cs = jnp.asarray(use_cs_ids, jnp.int32).reshape(1)
    cs32 = cs_ids.astype(jnp.int32)
    ck128, sel = _finish(pooled_ctx, pooled_know, cs32, use_cs)
    ck_attn = ck128[:, 0].reshape(N, K)

    ktok3 = know32.reshape(N * K, 1, TK)
    stok3 = src32.reshape(N, 1, TS)
    full_enc_flat, full_tok = _select(enc_know, enc_ctx, ktok3, stok3, sel)

    full_enc = full_enc_flat.reshape(N, TK + TS, D)
    full_mask = full_tok.reshape(N, TK + TS) != 0
    return full_enc, full_mask, ck_attn
